# baseline (speedup 1.0000x reference)
import jax
import jax.numpy as jnp
from jax import lax
from jax.experimental import pallas as pl
from jax.experimental.pallas import tpu as pltpu
from jax.experimental.pallas import tpu_sc as plsc

_B, _D, _P = 64, 96, 1024
_KSC = 16
_LANES = 16

_DG = 8
_GH = 3
_NS = 2
_SLICES = _DG * _P // _LANES


def _sc_body(x_hbm, p_hbm, o_hbm, pos_v, ibuf, obuf, psems, isems, osems):
    wid = lax.axis_index("s") * 2 + lax.axis_index("c")
    b0 = (wid // 4) * 2
    gh = wid % 4

    def d_off(g):
        return pl.multiple_of(gh * (_GH * _DG) + g * _DG, _DG)

    def pos_cp(g, s):
        return pltpu.make_async_copy(
            p_hbm.at[pl.ds(d_off(g), _DG)], pos_v.at[s], psems.at[s])

    def in_cp(g, s):
        return pltpu.make_async_copy(
            x_hbm.at[pl.ds(b0, 2), pl.ds(d_off(g), _DG)], ibuf.at[s],
            isems.at[s])

    def out_cp(g, s):
        return pltpu.make_async_copy(
            obuf.at[s], o_hbm.at[pl.ds(b0, 2), pl.ds(d_off(g), _DG)],
            osems.at[s])

    for s in range(_NS):
        pos_cp(s, s).start()
        in_cp(s, s).start()

    for g in range(_GH):
        s = g % _NS
        in_cp(g, s).wait()
        pos_cp(g, s).wait()
        if g >= _NS:
            out_cp(g - _NS, s).wait()

        def add_slices(j, carry, s=s):
            r = j // (_P // _LANES)
            col = (j % (_P // _LANES)) * _LANES
            sl = pl.ds(col, _LANES)
            pv = pos_v[s, r, sl]
            obuf[s, 0, r, sl] = ibuf[s, 0, r, sl] + pv
            obuf[s, 1, r, sl] = ibuf[s, 1, r, sl] + pv
            return carry

        lax.fori_loop(0, _SLICES, add_slices, 0, unroll=8)
        out_cp(g, s).start()
        if g + _NS < _GH:
            in_cp(g + _NS, s).start()
            pos_cp(g + _NS, s).start()

    for g in range(max(0, _GH - _NS), _GH):
        out_cp(g, g % _NS).wait()


def _run_sc(xt, pt):
    mesh = plsc.VectorSubcoreMesh(core_axis_name="c", subcore_axis_name="s")
    run = pl.kernel(
        _sc_body,
        mesh=mesh,
        out_type=jax.ShapeDtypeStruct((_KSC, _D, _P), jnp.float32),
        scratch_types=[
            pltpu.VMEM((_NS, _DG, _P), jnp.float32),
            pltpu.VMEM((_NS, 2, _DG, _P), jnp.float32),
            pltpu.VMEM((_NS, 2, _DG, _P), jnp.float32),
            pltpu.SemaphoreType.DMA((_NS,)),
            pltpu.SemaphoreType.DMA((_NS,)),
            pltpu.SemaphoreType.DMA((_NS,)),
        ],
    )
    return run(xt, pt)


_CB = 8
_NCH = (_B - _KSC) // _CB
_NBUF = 4


def _tc_body(x_hbm, p_ref, o_hbm, ibuf, obuf, isems, osems):
    pos = p_ref[...]

    def in_cp(c, s):
        return pltpu.make_async_copy(
            x_hbm.at[pl.ds(_KSC + c * _CB, _CB)], ibuf.at[s], isems.at[s])

    def out_cp(c, s):
        return pltpu.make_async_copy(
            obuf.at[s], o_hbm.at[pl.ds(_KSC + c * _CB, _CB)], osems.at[s])

    for s in range(min(_NBUF, _NCH)):
        in_cp(s, s).start()
    for c in range(_NCH):
        s = c % _NBUF
        in_cp(c, s).wait()
        if c >= _NBUF:
            out_cp(c - _NBUF, s).wait()
        obuf[s] = ibuf[s] + pos
        out_cp(c, s).start()
        if c + _NBUF < _NCH:
            in_cp(c + _NBUF, s).start()
    for c in range(max(0, _NCH - _NBUF), _NCH):
        out_cp(c, c % _NBUF).wait()


def _run_tc(xt, pt):
    return pl.pallas_call(
        _tc_body,
        in_specs=[
            pl.BlockSpec(memory_space=pl.ANY),
            pl.BlockSpec(memory_space=pltpu.MemorySpace.VMEM),
        ],
        out_specs=pl.BlockSpec(memory_space=pl.ANY),
        out_shape=jax.ShapeDtypeStruct((_B, _D, _P), jnp.float32),
        scratch_shapes=[
            pltpu.VMEM((_NBUF, _CB, _D, _P), jnp.float32),
            pltpu.VMEM((_NBUF, _CB, _D, _P), jnp.float32),
            pltpu.SemaphoreType.DMA((_NBUF,)),
            pltpu.SemaphoreType.DMA((_NBUF,)),
        ],
    )(xt, pt)


def kernel(x, pos_table):
    xt = jnp.swapaxes(x, 1, 2)
    pt = jnp.swapaxes(pos_table, 0, 1)
    sc_out = _run_sc(xt, pt)
    tc_out = _run_tc(xt, pt)
    out = lax.dynamic_update_slice(tc_out, sc_out, (0, 0, 0))
    return jnp.swapaxes(out, 1, 2)


# speedup vs baseline: 2.6277x; 2.6277x over previous
import jax
import jax.numpy as jnp
from jax.experimental import pallas as pl
from jax.experimental.pallas import tpu as pltpu

_B, _D, _P = 64, 96, 1024
_CB = 8
_NCH = _B // _CB
_NBUF = 4


def _add_body(x_hbm, p_ref, o_hbm, ibuf, obuf, isems, osems):
    pos = p_ref[...]

    def in_cp(c, s):
        return pltpu.make_async_copy(
            x_hbm.at[pl.ds(c * _CB, _CB)], ibuf.at[s], isems.at[s])

    def out_cp(c, s):
        return pltpu.make_async_copy(
            obuf.at[s], o_hbm.at[pl.ds(c * _CB, _CB)], osems.at[s])

    for s in range(_NBUF):
        in_cp(s, s).start()
    for c in range(_NCH):
        s = c % _NBUF
        in_cp(c, s).wait()
        if c >= _NBUF:
            out_cp(c - _NBUF, s).wait()
        obuf[s] = ibuf[s] + pos
        out_cp(c, s).start()
        if c + _NBUF < _NCH:
            in_cp(c + _NBUF, s).start()
    for c in range(_NCH - _NBUF, _NCH):
        out_cp(c, c % _NBUF).wait()


def kernel(x, pos_table):
    xt = jnp.swapaxes(x, 1, 2)
    pt = jnp.swapaxes(pos_table, 0, 1)
    out = pl.pallas_call(
        _add_body,
        in_specs=[
            pl.BlockSpec(memory_space=pl.ANY),
            pl.BlockSpec(memory_space=pltpu.MemorySpace.VMEM),
        ],
        out_specs=pl.BlockSpec(memory_space=pl.ANY),
        out_shape=jax.ShapeDtypeStruct((_B, _D, _P), jnp.float32),
        scratch_shapes=[
            pltpu.VMEM((_NBUF, _CB, _D, _P), jnp.float32),
            pltpu.VMEM((_NBUF, _CB, _D, _P), jnp.float32),
            pltpu.SemaphoreType.DMA((_NBUF,)),
            pltpu.SemaphoreType.DMA((_NBUF,)),
        ],
    )(xt, pt)
    return jnp.swapaxes(out, 1, 2)
